# baseline (speedup 1.0000x reference)
import math

import jax
import jax.numpy as jnp
from jax.experimental import pallas as pl

N = 170
B = 32
L = 12
GRAN = 288
KK = 136

F_CL = N * 102
F_FH = N * 18
F_C = N * L


def _leaky(x):
    return jnp.where(x > 0, x, 0.01 * x)


def _bdot(a, b):
    return jnp.dot(a, b, preferred_element_type=jnp.float32)


F_CL2 = N * 98


F_CLC = 34 * 102


def _cl_kernel(x_ref, w1_ref, b1_ref, w2_ref, b2_ref, o_ref):
    x = x_ref[0, 0]
    w1 = F_CLC - 4
    x2 = jnp.concatenate([x[dt:dt + w1] for dt in range(5)], axis=1)
    y1 = _bdot(x2, w1_ref[...])
    y1 = _leaky(y1 + b1_ref[0][None, :])
    w2 = w1 - 2
    z = _bdot(y1, w2_ref[...])
    y2 = z[0:w2, 0:12] + z[1:1 + w2, 12:24] + z[2:2 + w2, 24:36]
    y2 = jnp.tanh(y2 + b2_ref[0][None, :])
    o_ref[0, 0] = jnp.concatenate(
        [y2, jnp.zeros((6, y2.shape[1]), jnp.float32)], axis=0)


def _conv_pair_kernel(x_ref, w1_ref, b1_ref, w2_ref, b2_ref, o_ref):
    x = x_ref[0]
    w1 = F_FH - 4
    y1 = _bdot(x[0:w1], w1_ref[0])
    for dt in range(1, 5):
        y1 = y1 + _bdot(x[dt:dt + w1], w1_ref[dt])
    y1 = _leaky(y1 + b1_ref[0][None, :])
    w2 = w1 - 2
    y2 = _bdot(y1[0:w2], w2_ref[0])
    for dt in range(1, 3):
        y2 = y2 + _bdot(y1[dt:dt + w2], w2_ref[dt])
    y2 = jnp.tanh(y2 + b2_ref[0][None, :])
    o_ref[0] = jnp.concatenate(
        [y2, jnp.zeros((6, y2.shape[1]), jnp.float32)], axis=0)


def _tc_kernel(inp_ref, ed_ref, ew_ref, pw_ref, pb_ref,
               w1_ref, b1_ref, w2_ref, b2_ref, dgt_ref, dgb_ref,
               msel_ref, x0_ref, x1_ref, xg_ref, g_ref):
    xi = inp_ref[0]
    idx_d = (xi[:, 1:2] * GRAN).astype(jnp.int32)
    iota_d = jax.lax.broadcasted_iota(jnp.int32, (F_FH, GRAN), 1)
    oh_d = (iota_d == idx_d).astype(jnp.float32)
    td = jnp.dot(oh_d, ed_ref[...], preferred_element_type=jnp.float32)
    idx_w = xi[:, 2:3].astype(jnp.int32)
    iota_w = jax.lax.broadcasted_iota(jnp.int32, (F_FH, 8), 1)
    oh_w = (iota_w == idx_w).astype(jnp.float32)
    tw = jnp.dot(oh_w, ew_ref[...], preferred_element_type=jnp.float32)
    proj = jnp.dot(xi, pw_ref[...], preferred_element_type=jnp.float32)
    proj = proj + pb_ref[0][None, :]
    x = jnp.concatenate([proj, td, tw], axis=1)
    x0_ref[0] = x
    w1 = F_FH - 4
    y1 = jnp.dot(x[0:w1], w1_ref[0], preferred_element_type=jnp.float32)
    for dt in range(1, 5):
        y1 = y1 + jnp.dot(x[dt:dt + w1], w1_ref[dt],
                          preferred_element_type=jnp.float32)
    y1 = _leaky(y1 + b1_ref[0][None, :])
    w2 = w1 - 2
    y2 = jnp.dot(y1[0:w2], w2_ref[0], preferred_element_type=jnp.float32)
    for dt in range(1, 3):
        y2 = y2 + jnp.dot(y1[dt:dt + w2], w2_ref[dt],
                          preferred_element_type=jnp.float32)
    y2 = jnp.tanh(y2 + b2_ref[0][None, :])
    pad = jnp.zeros((6, 128), jnp.float32)
    x1 = jnp.concatenate([y2, pad], axis=0)
    x1_ref[0] = x1
    xg = jnp.dot(y2, dgt_ref[...], preferred_element_type=jnp.float32)
    xg = xg + dgb_ref[0][None, :]
    xgp = jnp.concatenate([xg, pad], axis=0)
    xg_ref[0] = xgp
    rs = jnp.sum(xgp, axis=1, keepdims=True)
    g_ref[0] = jnp.dot(msel_ref[...], rs, preferred_element_type=jnp.float32)


def _emb_kernel(x_ref, ed_ref, ew_ref, pw_ref, pb_ref, o_ref):
    x = x_ref[0]
    idx_d = (x[:, 1:2] * GRAN).astype(jnp.int32)
    iota_d = jax.lax.broadcasted_iota(jnp.int32, (F_FH, GRAN), 1)
    oh_d = (iota_d == idx_d).astype(jnp.float32)
    td = jnp.dot(oh_d, ed_ref[...], preferred_element_type=jnp.float32)
    idx_w = x[:, 2:3].astype(jnp.int32)
    iota_w = jax.lax.broadcasted_iota(jnp.int32, (F_FH, 8), 1)
    oh_w = (iota_w == idx_w).astype(jnp.float32)
    tw = jnp.dot(oh_w, ew_ref[...], preferred_element_type=jnp.float32)
    proj = jnp.dot(x, pw_ref[...], preferred_element_type=jnp.float32)
    proj = proj + pb_ref[0][None, :]
    o_ref[0] = jnp.concatenate([proj, td, tw], axis=1)


def _t_kernel(mem_ref, o_ref):
    mem = mem_ref[...]
    s = jax.lax.dot_general(mem, mem, (((0,), (0,)), ((), ())),
                            preferred_element_type=jnp.float32)
    s = jnp.maximum(s, 0.0)
    r = jax.lax.broadcasted_iota(jnp.int32, (N, N), 0)
    c = jax.lax.broadcasted_iota(jnp.int32, (N, N), 1)
    eye = (r == c)
    s = jnp.where(eye, 0.1, s)
    m = jnp.max(s, axis=1, keepdims=True)
    e = jnp.exp(s - m)
    sw = e / jnp.sum(e, axis=1, keepdims=True)
    eye_f = eye.astype(jnp.float32)

    def cond(carry):
        _acc, _p, k, fixed = carry
        return jnp.logical_and(k < N, jnp.logical_not(fixed))

    def body(carry):
        acc, p, k, _fixed = carry
        nxt = jnp.dot(sw, p, preferred_element_type=jnp.float32)
        return acc + nxt, nxt, k + 1, jnp.all(nxt == p)

    acc, p, k, _f = jax.lax.while_loop(
        cond, body, (eye_f, eye_f, jnp.int32(0), jnp.bool_(False)))

    def tail(_, a):
        return a + p

    acc = jax.lax.fori_loop(k, N, tail, acc)
    o_ref[...] = acc


def _af_kernel(g_ref, trep_ref, o_ref):
    gsum = g_ref[...]
    a0 = jnp.maximum(gsum * trep_ref[...] * (1.0 / math.sqrt(128.0)), 0.0)
    m1 = jnp.max(a0, axis=1, keepdims=True)
    e1 = jnp.exp(a0 - m1)
    ap = e1 / jnp.sum(e1, axis=1, keepdims=True)
    m2 = jnp.max(ap, axis=1, keepdims=True)
    e2 = jnp.exp(ap - m2)
    af = e2 / jnp.sum(e2, axis=1, keepdims=True)

    afb = jax.lax.bitcast_convert_type(af, jnp.int32)
    lo0 = jnp.min(afb, axis=1, keepdims=True)
    hi0 = jnp.max(afb, axis=1, keepdims=True) + 1

    def body(_, carry):
        lo, hi = carry
        mid = lo + jax.lax.shift_right_arithmetic(hi - lo, 1)
        cnt = jnp.sum((afb >= mid).astype(jnp.int32), axis=1, keepdims=True)
        pred = cnt >= KK
        return jnp.where(pred, mid, lo), jnp.where(pred, hi, mid)

    lo, _hi = jax.lax.fori_loop(0, 31, body, (lo0, hi0))
    gt = afb > lo
    cnt_gt = jnp.sum(gt.astype(jnp.float32), axis=1, keepdims=True)
    need_eq = float(KK) - cnt_gt
    eq = (afb == lo).astype(jnp.float32)
    rr = jax.lax.broadcasted_iota(jnp.int32, (N, N), 0)
    cc = jax.lax.broadcasted_iota(jnp.int32, (N, N), 1)
    lt = (rr < cc).astype(jnp.float32)
    eq_rank = jnp.dot(eq, lt, preferred_element_type=jnp.float32)
    keep_eq = (eq > 0.5) & (eq_rank < need_eq)
    mask = gt | keep_eq
    o_ref[...] = af * mask.astype(jnp.float32)


def _diff_kernel(af_ref, xg3_ref, o1_ref, o2_ref):
    af = af_ref[0]
    xg3 = xg3_ref[0]
    dn = (((0,), (0,)), ((), ()))
    o1 = jax.lax.dot_general(af, xg3, dn, preferred_element_type=jnp.float32)
    o1_ref[0] = o1
    o2_ref[0] = jax.lax.dot_general(af, o1, dn,
                                    preferred_element_type=jnp.float32)


def _glu_kernel(o1_ref, o2_ref, skip_ref, x0_ref, hst_ref, lib_ref,
                wat_ref, wbt_ref, gb_ref, w1_ref, b1_ref, w2_ref, b2_ref,
                w3_ref, b3_ref, o_ref):
    o1 = o1_ref[0]
    o2 = o2_ref[0]
    xc = (_bdot(o1, wat_ref[...]) + _bdot(o2, wbt_ref[...])
          + gb_ref[0][None, :])
    dout = xc * lib_ref[...] + skip_ref[0]
    x0 = x0_ref[0]
    x0s = jnp.concatenate(
        [x0[3:], jnp.zeros((3, x0.shape[1]), jnp.float32)], axis=0)
    xx = dout + x0s + hst_ref[0]
    g1 = _bdot(xx, w1_ref[...]) + b1_ref[0][None, :]
    g2 = _bdot(xx, w2_ref[...]) + b2_ref[0][None, :]
    gg = _bdot(g1 * jax.nn.sigmoid(g2), w3_ref[...]) + b3_ref[0][None, :]
    o_ref[0] = jnp.maximum(gg + xx, 0.0)


def _reg_kernel(rx_ref, w_ref, b_ref, o_ref):
    o_ref[...] = (jnp.dot(rx_ref[...], w_ref[...],
                          preferred_element_type=jnp.float32)
                  + b_ref[0][None, :])


def _full(shape):
    nd = len(shape)
    return pl.BlockSpec(shape, lambda b, _n=nd: (0,) * _n)


def _batched(shape):
    nd = len(shape)
    return pl.BlockSpec((1,) + shape, lambda b, _n=nd: (b,) + (0,) * _n)


def kernel(input, hidden_states, cl_w1, cl_b1, cl_w2, cl_b2, fh_w1, fh_b1,
           fh_w2, fh_b2, emb_day, emb_week, proj_w, proj_b, tc_w1, tc_b1,
           tc_w2, tc_b2, dg_w, dg_b, mem, gcn_w, gcn_b, lib, glu_w1, glu_b1,
           glu_w2, glu_b2, glu_w3, glu_b3, reg_w, reg_b):
    f32 = jnp.float32

    w_cl1 = jnp.transpose(cl_w1[:, :, 0, :], (2, 1, 0)).reshape(40, 64)
    w_cl2 = jnp.transpose(cl_w2[:, :, 0, :], (1, 2, 0)).reshape(64, 36)
    w_fh1 = jnp.transpose(fh_w1[:, :, 0, :], (2, 1, 0))
    w_fh2 = jnp.transpose(fh_w2[:, :, 0, :], (2, 1, 0))
    w_tc1 = jnp.transpose(tc_w1[:, :, 0, :], (2, 1, 0))
    w_tc2 = jnp.transpose(tc_w2[:, :, 0, :], (2, 1, 0))
    dgt = dg_w[:, :, 0, 0].T
    wat = gcn_w[:, :128, 0, 0].T
    wbt = gcn_w[:, 128:, 0, 0].T
    g1t, g2t, g3t = glu_w1.T, glu_w2.T, glu_w3.T
    wreg = jnp.transpose(reg_w[:, :, 0, :], (2, 1, 0)).reshape(128 * L, 12)
    ew_pad = jnp.concatenate([emb_week, jnp.zeros((1, 32), f32)], axis=0)
    pwt = proj_w.T
    lib_rp = jnp.pad(jnp.transpose(lib, (1, 2, 0)),
                     ((0, 0), (0, 6), (0, 0))).reshape(F_FH, 128)
    b2d = lambda v: v[None, :]

    t_mat = pl.pallas_call(
        _t_kernel,
        out_shape=jax.ShapeDtypeStruct((N, N), f32),
    )(mem)
    t_rep = jnp.tile(t_mat, (B, 1))

    hsp = jnp.pad(hidden_states, ((0, 0), (0, 0), (0, 0), (3, 3)),
                  mode='edge')
    hsp_t = jnp.transpose(hsp, (0, 1, 3, 2)).reshape(B, 5, F_CLC, 8)
    h_cl = pl.pallas_call(
        _cl_kernel,
        grid=(B, 5),
        in_specs=[pl.BlockSpec((1, 1, F_CLC, 8), lambda b, c: (b, c, 0, 0)),
                  pl.BlockSpec((40, 64), lambda b, c: (0, 0)),
                  pl.BlockSpec((1, 64), lambda b, c: (0, 0)),
                  pl.BlockSpec((64, 36), lambda b, c: (0, 0)),
                  pl.BlockSpec((1, 12), lambda b, c: (0, 0))],
        out_specs=pl.BlockSpec((1, 1, F_CLC, 12), lambda b, c: (b, c, 0, 0)),
        out_shape=jax.ShapeDtypeStruct((B, 5, F_CLC, 12), f32),
    )(hsp_t, w_cl1, b2d(cl_b1), w_cl2, b2d(cl_b2))
    h_cl = h_cl.reshape(B, F_CL, 12)

    fh_in = h_cl.reshape(B, N, 102, 12)[:, :, :96, :]
    fh_in = jnp.pad(jnp.transpose(fh_in, (0, 1, 3, 2)),
                    ((0, 0), (0, 0), (3, 3), (0, 0)),
                    mode='edge').reshape(B, F_FH, 96)
    h_fh = pl.pallas_call(
        _conv_pair_kernel,
        grid=(B,),
        in_specs=[_batched((F_FH, 96)), _full((5, 96, 258)), _full((1, 258)),
                  _full((3, 258, 128)), _full((1, 128))],
        out_specs=_batched((F_FH, 128)),
        out_shape=jax.ShapeDtypeStruct((B, F_FH, 128), f32),
    )(fh_in, w_fh1, b2d(fh_b1), w_fh2, b2d(fh_b2))

    inp_r = jnp.pad(jnp.transpose(input, (0, 2, 1, 3)),
                    ((0, 0), (0, 0), (3, 3), (0, 0)),
                    mode='edge').reshape(B, F_FH, 3)
    rows = jnp.arange(F_FH)
    msel = (((rows[None, :] // 18) == jnp.arange(N)[:, None])
            & ((rows[None, :] % 18) < L)).astype(f32)
    x0p, x1_full, xg_full, g_out = pl.pallas_call(
        _tc_kernel,
        grid=(B,),
        in_specs=[_batched((F_FH, 3)), _full((GRAN, 32)), _full((8, 32)),
                  _full((3, 64)), _full((1, 64)),
                  _full((5, 128, 128)),
                  _full((1, 128)), _full((3, 128, 128)), _full((1, 128)),
                  _full((128, 128)), _full((1, 128)), _full((N, F_FH))],
        out_specs=(_batched((F_FH, 128)), _batched((F_FH, 128)),
                   _batched((F_FH, 128)), _batched((N, 1))),
        out_shape=(jax.ShapeDtypeStruct((B, F_FH, 128), f32),
                   jax.ShapeDtypeStruct((B, F_FH, 128), f32),
                   jax.ShapeDtypeStruct((B, F_FH, 128), f32),
                   jax.ShapeDtypeStruct((B, N, 1), f32)),
    )(inp_r, emb_day, ew_pad, pwt, b2d(proj_b),
      w_tc1, b2d(tc_b1), w_tc2, b2d(tc_b2), dgt, b2d(dg_b), msel)
    xg3p = xg_full.reshape(B, N, 18 * 128)

    af = pl.pallas_call(
        _af_kernel,
        out_shape=jax.ShapeDtypeStruct((B * N, N), f32),
    )(g_out.reshape(B * N, 1), t_rep)
    a_b = af.reshape(B, N, N)

    o1, o2 = pl.pallas_call(
        _diff_kernel,
        grid=(B,),
        in_specs=[_batched((N, N)), _batched((N, 18 * 128))],
        out_specs=(_batched((N, 18 * 128)), _batched((N, 18 * 128))),
        out_shape=(jax.ShapeDtypeStruct((B, N, 18 * 128), f32),
                   jax.ShapeDtypeStruct((B, N, 18 * 128), f32)),
    )(a_b, xg3p)
    o1r = o1.reshape(B, F_FH, 128)
    o2r = o2.reshape(B, F_FH, 128)

    rx = pl.pallas_call(
        _glu_kernel,
        grid=(B,),
        in_specs=[_batched((F_FH, 128)), _batched((F_FH, 128)),
                  _batched((F_FH, 128)), _batched((F_FH, 128)),
                  _batched((F_FH, 128)), _full((F_FH, 128)),
                  _full((128, 128)), _full((128, 128)), _full((1, 128)),
                  _full((128, 128)), _full((1, 128)),
                  _full((128, 128)), _full((1, 128)),
                  _full((128, 128)), _full((1, 128))],
        out_specs=_batched((F_FH, 128)),
        out_shape=jax.ShapeDtypeStruct((B, F_FH, 128), f32),
    )(o1r, o2r, x1_full, x0p, h_fh, lib_rp, wat, wbt, b2d(gcn_b),
      g1t, b2d(glu_b1), g2t, b2d(glu_b2), g3t, b2d(glu_b3))

    rx3 = rx.reshape(B, N, 18 * 128)[:, :, :128 * L]
    pred = pl.pallas_call(
        _reg_kernel,
        out_shape=jax.ShapeDtypeStruct((B * N, 12), f32),
    )(rx3.reshape(B * N, 128 * L), wreg, b2d(reg_b))
    return jnp.transpose(pred.reshape(B, N, 12), (0, 2, 1))[..., None]


# speedup vs baseline: 1.0819x; 1.0819x over previous
import math

import jax
import jax.numpy as jnp
from jax.experimental import pallas as pl

N = 170
B = 32
L = 12
GRAN = 288
KK = 136

F_CL = N * 102
F_FH = N * 18
F_C = N * L


def _leaky(x):
    return jnp.where(x > 0, x, 0.01 * x)


def _bdot(a, b):
    return jnp.dot(a, b, preferred_element_type=jnp.float32)


F_CL2 = N * 98


def _cl_kernel(x_ref, w1_ref, b1_ref, w2_ref, b2_ref, o_ref):
    x = x_ref[0]
    y1 = _bdot(x, w1_ref[...])
    y1 = _leaky(y1 + b1_ref[0][None, :])
    w2 = F_CL2 - 2
    z = _bdot(y1, w2_ref[...])
    y2 = z[0:w2, 0:12] + z[1:1 + w2, 12:24] + z[2:2 + w2, 24:36]
    y2 = jnp.tanh(y2 + b2_ref[0][None, :])
    o_ref[0] = jnp.concatenate(
        [y2, jnp.zeros((2, y2.shape[1]), jnp.float32)], axis=0)


def _conv_pair_kernel(x_ref, w1_ref, b1_ref, w2_ref, b2_ref, o_ref):
    x = x_ref[0]
    w1 = F_FH - 4
    y1 = _bdot(x[0:w1], w1_ref[0])
    for dt in range(1, 5):
        y1 = y1 + _bdot(x[dt:dt + w1], w1_ref[dt])
    y1 = _leaky(y1 + b1_ref[0][None, :])
    w2 = w1 - 2
    y2 = _bdot(y1[0:w2], w2_ref[0])
    for dt in range(1, 3):
        y2 = y2 + _bdot(y1[dt:dt + w2], w2_ref[dt])
    y2 = jnp.tanh(y2 + b2_ref[0][None, :])
    o_ref[0] = jnp.concatenate(
        [y2, jnp.zeros((6, y2.shape[1]), jnp.float32)], axis=0)


def _tc_kernel(inp_ref, ed_ref, ew_ref, pw_ref, pb_ref,
               w1_ref, b1_ref, w2_ref, b2_ref, dgt_ref, dgb_ref,
               msel_ref, x0_ref, x1_ref, xg_ref, g_ref):
    xi = inp_ref[0]
    idx_d = (xi[:, 1:2] * GRAN).astype(jnp.int32)
    iota_d = jax.lax.broadcasted_iota(jnp.int32, (F_FH, GRAN), 1)
    oh_d = (iota_d == idx_d).astype(jnp.float32)
    td = jnp.dot(oh_d, ed_ref[...], preferred_element_type=jnp.float32)
    idx_w = xi[:, 2:3].astype(jnp.int32)
    iota_w = jax.lax.broadcasted_iota(jnp.int32, (F_FH, 8), 1)
    oh_w = (iota_w == idx_w).astype(jnp.float32)
    tw = jnp.dot(oh_w, ew_ref[...], preferred_element_type=jnp.float32)
    proj = jnp.dot(xi, pw_ref[...], preferred_element_type=jnp.float32)
    proj = proj + pb_ref[0][None, :]
    x = jnp.concatenate([proj, td, tw], axis=1)
    x0_ref[0] = x
    w1 = F_FH - 4
    y1 = jnp.dot(x[0:w1], w1_ref[0], preferred_element_type=jnp.float32)
    for dt in range(1, 5):
        y1 = y1 + jnp.dot(x[dt:dt + w1], w1_ref[dt],
                          preferred_element_type=jnp.float32)
    y1 = _leaky(y1 + b1_ref[0][None, :])
    w2 = w1 - 2
    y2 = jnp.dot(y1[0:w2], w2_ref[0], preferred_element_type=jnp.float32)
    for dt in range(1, 3):
        y2 = y2 + jnp.dot(y1[dt:dt + w2], w2_ref[dt],
                          preferred_element_type=jnp.float32)
    y2 = jnp.tanh(y2 + b2_ref[0][None, :])
    pad = jnp.zeros((6, 128), jnp.float32)
    x1 = jnp.concatenate([y2, pad], axis=0)
    x1_ref[0] = x1
    xg = jnp.dot(y2, dgt_ref[...], preferred_element_type=jnp.float32)
    xg = xg + dgb_ref[0][None, :]
    xgp = jnp.concatenate([xg, pad], axis=0)
    xg_ref[0] = xgp
    rs = jnp.sum(xgp, axis=1, keepdims=True)
    g_ref[0] = jnp.dot(msel_ref[...], rs, preferred_element_type=jnp.float32)


def _emb_kernel(x_ref, ed_ref, ew_ref, pw_ref, pb_ref, o_ref):
    x = x_ref[0]
    idx_d = (x[:, 1:2] * GRAN).astype(jnp.int32)
    iota_d = jax.lax.broadcasted_iota(jnp.int32, (F_FH, GRAN), 1)
    oh_d = (iota_d == idx_d).astype(jnp.float32)
    td = jnp.dot(oh_d, ed_ref[...], preferred_element_type=jnp.float32)
    idx_w = x[:, 2:3].astype(jnp.int32)
    iota_w = jax.lax.broadcasted_iota(jnp.int32, (F_FH, 8), 1)
    oh_w = (iota_w == idx_w).astype(jnp.float32)
    tw = jnp.dot(oh_w, ew_ref[...], preferred_element_type=jnp.float32)
    proj = jnp.dot(x, pw_ref[...], preferred_element_type=jnp.float32)
    proj = proj + pb_ref[0][None, :]
    o_ref[0] = jnp.concatenate([proj, td, tw], axis=1)


def _t_kernel(mem_ref, o_ref):
    mem = mem_ref[...]
    s = jax.lax.dot_general(mem, mem, (((0,), (0,)), ((), ())),
                            preferred_element_type=jnp.float32)
    s = jnp.maximum(s, 0.0)
    r = jax.lax.broadcasted_iota(jnp.int32, (N, N), 0)
    c = jax.lax.broadcasted_iota(jnp.int32, (N, N), 1)
    eye = (r == c)
    s = jnp.where(eye, 0.1, s)
    m = jnp.max(s, axis=1, keepdims=True)
    e = jnp.exp(s - m)
    sw = e / jnp.sum(e, axis=1, keepdims=True)
    eye_f = eye.astype(jnp.float32)

    def cond(carry):
        _acc, _p, k, fixed = carry
        return jnp.logical_and(k < N, jnp.logical_not(fixed))

    def body(carry):
        acc, p, k, _fixed = carry
        nxt = jnp.dot(sw, p, preferred_element_type=jnp.float32)
        return acc + nxt, nxt, k + 1, jnp.all(nxt == p)

    acc, p, k, _f = jax.lax.while_loop(
        cond, body, (eye_f, eye_f, jnp.int32(0), jnp.bool_(False)))

    def tail(_, a):
        return a + p

    acc = jax.lax.fori_loop(k, N, tail, acc)
    o_ref[...] = acc


def _af_kernel(g_ref, trep_ref, o_ref):
    gsum = g_ref[...]
    a0 = jnp.maximum(gsum * trep_ref[...] * (1.0 / math.sqrt(128.0)), 0.0)
    m1 = jnp.max(a0, axis=1, keepdims=True)
    e1 = jnp.exp(a0 - m1)
    ap = e1 / jnp.sum(e1, axis=1, keepdims=True)
    m2 = jnp.max(ap, axis=1, keepdims=True)
    e2 = jnp.exp(ap - m2)
    af = e2 / jnp.sum(e2, axis=1, keepdims=True)

    afb = jax.lax.bitcast_convert_type(af, jnp.int32)
    lo0 = jnp.min(afb, axis=1, keepdims=True)
    hi0 = jnp.max(afb, axis=1, keepdims=True) + 1

    def body(_, carry):
        lo, hi = carry
        mid = lo + jax.lax.shift_right_arithmetic(hi - lo, 1)
        cnt = jnp.sum((afb >= mid).astype(jnp.int32), axis=1, keepdims=True)
        pred = cnt >= KK
        return jnp.where(pred, mid, lo), jnp.where(pred, hi, mid)

    lo, _hi = jax.lax.fori_loop(0, 31, body, (lo0, hi0))
    gt = afb > lo
    cnt_gt = jnp.sum(gt.astype(jnp.float32), axis=1, keepdims=True)
    need_eq = float(KK) - cnt_gt
    eq = (afb == lo).astype(jnp.float32)
    rr = jax.lax.broadcasted_iota(jnp.int32, (N, N), 0)
    cc = jax.lax.broadcasted_iota(jnp.int32, (N, N), 1)
    lt = (rr < cc).astype(jnp.float32)
    eq_rank = jnp.dot(eq, lt, preferred_element_type=jnp.float32)
    keep_eq = (eq > 0.5) & (eq_rank < need_eq)
    mask = gt | keep_eq
    o_ref[...] = af * mask.astype(jnp.float32)


def _diff_kernel(af_ref, xg3_ref, o1_ref, o2_ref):
    af = af_ref[0]
    xg3 = xg3_ref[0]
    dn = (((0,), (0,)), ((), ()))
    o1 = jax.lax.dot_general(af, xg3, dn, preferred_element_type=jnp.float32)
    o1_ref[0] = o1
    o2_ref[0] = jax.lax.dot_general(af, o1, dn,
                                    preferred_element_type=jnp.float32)


def _glu_kernel(o1_ref, o2_ref, skip_ref, x0_ref, hst_ref, lib_ref,
                wat_ref, wbt_ref, gb_ref, w1_ref, b1_ref, w2_ref, b2_ref,
                w3_ref, b3_ref, o_ref):
    o1 = o1_ref[0]
    o2 = o2_ref[0]
    xc = (_bdot(o1, wat_ref[...]) + _bdot(o2, wbt_ref[...])
          + gb_ref[0][None, :])
    dout = xc * lib_ref[...] + skip_ref[0]
    x0 = x0_ref[0]
    x0s = jnp.concatenate(
        [x0[3:], jnp.zeros((3, x0.shape[1]), jnp.float32)], axis=0)
    xx = dout + x0s + hst_ref[0]
    g1 = _bdot(xx, w1_ref[...]) + b1_ref[0][None, :]
    g2 = _bdot(xx, w2_ref[...]) + b2_ref[0][None, :]
    gg = _bdot(g1 * jax.nn.sigmoid(g2), w3_ref[...]) + b3_ref[0][None, :]
    o_ref[0] = jnp.maximum(gg + xx, 0.0)


def _reg_kernel(rx_ref, w_ref, b_ref, o_ref):
    o_ref[...] = (jnp.dot(rx_ref[...], w_ref[...],
                          preferred_element_type=jnp.float32)
                  + b_ref[0][None, :])


def _full(shape):
    nd = len(shape)
    return pl.BlockSpec(shape, lambda b, _n=nd: (0,) * _n)


def _batched(shape):
    nd = len(shape)
    return pl.BlockSpec((1,) + shape, lambda b, _n=nd: (b,) + (0,) * _n)


def kernel(input, hidden_states, cl_w1, cl_b1, cl_w2, cl_b2, fh_w1, fh_b1,
           fh_w2, fh_b2, emb_day, emb_week, proj_w, proj_b, tc_w1, tc_b1,
           tc_w2, tc_b2, dg_w, dg_b, mem, gcn_w, gcn_b, lib, glu_w1, glu_b1,
           glu_w2, glu_b2, glu_w3, glu_b3, reg_w, reg_b):
    f32 = jnp.float32

    w_cl1 = jnp.transpose(cl_w1[:, :, 0, :], (2, 1, 0)).reshape(40, 64)
    w_cl2 = jnp.transpose(cl_w2[:, :, 0, :], (1, 2, 0)).reshape(64, 36)
    w_fh1 = jnp.transpose(fh_w1[:, :, 0, :], (2, 1, 0))
    w_fh2 = jnp.transpose(fh_w2[:, :, 0, :], (2, 1, 0))
    w_tc1 = jnp.transpose(tc_w1[:, :, 0, :], (2, 1, 0))
    w_tc2 = jnp.transpose(tc_w2[:, :, 0, :], (2, 1, 0))
    dgt = dg_w[:, :, 0, 0].T
    wat = gcn_w[:, :128, 0, 0].T
    wbt = gcn_w[:, 128:, 0, 0].T
    g1t, g2t, g3t = glu_w1.T, glu_w2.T, glu_w3.T
    wreg = jnp.transpose(reg_w[:, :, 0, :], (2, 1, 0)).reshape(128 * L, 12)
    ew_pad = jnp.concatenate([emb_week, jnp.zeros((1, 32), f32)], axis=0)
    pwt = proj_w.T
    lib_rp = jnp.pad(jnp.transpose(lib, (1, 2, 0)),
                     ((0, 0), (0, 6), (0, 0))).reshape(F_FH, 128)
    b2d = lambda v: v[None, :]

    t_mat = pl.pallas_call(
        _t_kernel,
        out_shape=jax.ShapeDtypeStruct((N, N), f32),
    )(mem)
    t_rep = jnp.tile(t_mat, (B, 1))

    hsp = jnp.pad(hidden_states, ((0, 0), (0, 0), (0, 0), (3, 3)),
                  mode='edge')
    hsp_t = jnp.transpose(hsp, (0, 1, 3, 2))
    x2 = jnp.concatenate([hsp_t[:, :, dt:dt + 98, :] for dt in range(5)],
                         axis=-1).reshape(B, F_CL2, 40)
    h_cl = pl.pallas_call(
        _cl_kernel,
        grid=(B,),
        in_specs=[_batched((F_CL2, 40)), _full((40, 64)), _full((1, 64)),
                  _full((64, 36)), _full((1, 12))],
        out_specs=_batched((F_CL2, 12)),
        out_shape=jax.ShapeDtypeStruct((B, F_CL2, 12), f32),
    )(x2, w_cl1, b2d(cl_b1), w_cl2, b2d(cl_b2))

    fh_in = h_cl.reshape(B, N, 98, 12)[:, :, :96, :]
    fh_in = jnp.pad(jnp.transpose(fh_in, (0, 1, 3, 2)),
                    ((0, 0), (0, 0), (3, 3), (0, 0)),
                    mode='edge').reshape(B, F_FH, 96)
    h_fh = pl.pallas_call(
        _conv_pair_kernel,
        grid=(B,),
        in_specs=[_batched((F_FH, 96)), _full((5, 96, 258)), _full((1, 258)),
                  _full((3, 258, 128)), _full((1, 128))],
        out_specs=_batched((F_FH, 128)),
        out_shape=jax.ShapeDtypeStruct((B, F_FH, 128), f32),
    )(fh_in, w_fh1, b2d(fh_b1), w_fh2, b2d(fh_b2))

    inp_r = jnp.pad(jnp.transpose(input, (0, 2, 1, 3)),
                    ((0, 0), (0, 0), (3, 3), (0, 0)),
                    mode='edge').reshape(B, F_FH, 3)
    rows = jnp.arange(F_FH)
    msel = (((rows[None, :] // 18) == jnp.arange(N)[:, None])
            & ((rows[None, :] % 18) < L)).astype(f32)
    x0p, x1_full, xg_full, g_out = pl.pallas_call(
        _tc_kernel,
        grid=(B,),
        in_specs=[_batched((F_FH, 3)), _full((GRAN, 32)), _full((8, 32)),
                  _full((3, 64)), _full((1, 64)),
                  _full((5, 128, 128)),
                  _full((1, 128)), _full((3, 128, 128)), _full((1, 128)),
                  _full((128, 128)), _full((1, 128)), _full((N, F_FH))],
        out_specs=(_batched((F_FH, 128)), _batched((F_FH, 128)),
                   _batched((F_FH, 128)), _batched((N, 1))),
        out_shape=(jax.ShapeDtypeStruct((B, F_FH, 128), f32),
                   jax.ShapeDtypeStruct((B, F_FH, 128), f32),
                   jax.ShapeDtypeStruct((B, F_FH, 128), f32),
                   jax.ShapeDtypeStruct((B, N, 1), f32)),
    )(inp_r, emb_day, ew_pad, pwt, b2d(proj_b),
      w_tc1, b2d(tc_b1), w_tc2, b2d(tc_b2), dgt, b2d(dg_b), msel)
    xg3p = xg_full.reshape(B, N, 18 * 128)

    af = pl.pallas_call(
        _af_kernel,
        out_shape=jax.ShapeDtypeStruct((B * N, N), f32),
    )(g_out.reshape(B * N, 1), t_rep)
    a_b = af.reshape(B, N, N)

    o1, o2 = pl.pallas_call(
        _diff_kernel,
        grid=(B,),
        in_specs=[_batched((N, N)), _batched((N, 18 * 128))],
        out_specs=(_batched((N, 18 * 128)), _batched((N, 18 * 128))),
        out_shape=(jax.ShapeDtypeStruct((B, N, 18 * 128), f32),
                   jax.ShapeDtypeStruct((B, N, 18 * 128), f32)),
    )(a_b, xg3p)
    o1r = o1.reshape(B, F_FH, 128)
    o2r = o2.reshape(B, F_FH, 128)

    rx = pl.pallas_call(
        _glu_kernel,
        grid=(B,),
        in_specs=[_batched((F_FH, 128)), _batched((F_FH, 128)),
                  _batched((F_FH, 128)), _batched((F_FH, 128)),
                  _batched((F_FH, 128)), _full((F_FH, 128)),
                  _full((128, 128)), _full((128, 128)), _full((1, 128)),
                  _full((128, 128)), _full((1, 128)),
                  _full((128, 128)), _full((1, 128)),
                  _full((128, 128)), _full((1, 128))],
        out_specs=_batched((F_FH, 128)),
        out_shape=jax.ShapeDtypeStruct((B, F_FH, 128), f32),
    )(o1r, o2r, x1_full, x0p, h_fh, lib_rp, wat, wbt, b2d(gcn_b),
      g1t, b2d(glu_b1), g2t, b2d(glu_b2), g3t, b2d(glu_b3))

    rx3 = rx.reshape(B, N, 18 * 128)[:, :, :128 * L]
    pred = pl.pallas_call(
        _reg_kernel,
        out_shape=jax.ShapeDtypeStruct((B * N, 12), f32),
    )(rx3.reshape(B * N, 128 * L), wreg, b2d(reg_b))
    return jnp.transpose(pred.reshape(B, N, 12), (0, 2, 1))[..., None]
